# baseline (speedup 1.0000x reference)
import jax
import jax.numpy as jnp
from jax import lax
from jax.experimental import pallas as pl
from jax.experimental.pallas import tpu as pltpu
from jax.experimental.pallas import tpu_sc as plsc

NSEG = 1024
LANES = 16
NC = 2
NS = 16
NW = NC * NS


def _make_sc_kernel(N, D, C, interpret=False):
    SPW = NSEG // NW
    DF = 3 * D
    NCH = DF // LANES
    DCH = D // LANES
    SSTR = 64
    NSAMP = (N + SSTR - 1) // SSTR
    CSTEPS = NSAMP.bit_length()
    WSTEPS = (SSTR // LANES).bit_length()
    assert N % LANES == 0 and C % 8 == 0 and (N - C) % 8 == 0

    def body(h0, h1, h2, idxh, samph, out,
             samp, win, b0, b1, b2, acc, bnd, sem0, sem1, sem2, sem3, semw):
        sems = (sem0, sem1, sem2, sem3)
        bufs = (b0, b1, b2)
        cid = lax.axis_index("c")
        sid = lax.axis_index("s")
        w = sid * NC + cid
        seg_lo = w * SPW

        pltpu.sync_copy(samph, samp.at[pl.ds(0, NSAMP)])

        def coarse(x):
            def bisect(_, carry):
                lo_b, hi_b = carry
                active = lo_b < hi_b
                mid = (lo_b + hi_b) // 2
                v = samp[pl.ds(mid, LANES)][0]
                lt = jnp.logical_and(active, v < x)
                ge = jnp.logical_and(active, jnp.logical_not(v < x))
                lo_b = jnp.where(lt, mid + 1, lo_b)
                hi_b = jnp.where(ge, mid, hi_b)
                return lo_b, hi_b

            p, _ = lax.fori_loop(0, CSTEPS, bisect,
                                 (jnp.int32(0), jnp.int32(NSAMP)))
            return jnp.minimum(jnp.maximum(p - 1, 0) * SSTR, N - SSTR)

        def fire_body(s, _):
            wb = coarse(seg_lo + s)
            bnd[s] = wb
            pltpu.async_copy(idxh.at[pl.ds(wb, SSTR)],
                             win.at[pl.ds(s * SSTR, SSTR)], semw)
            return 0

        def drain_body(s, _):
            pltpu.make_async_copy(idxh.at[pl.ds(0, SSTR)],
                                  win.at[pl.ds(s * SSTR, SSTR)], semw).wait()
            return 0

        def refine_body(s, _):
            x = seg_lo + s

            def wbisect(_, carry):
                lo_b, hi_b = carry
                active = lo_b < hi_b
                mid = (lo_b + hi_b) // 2
                v = win[pl.ds(s * SSTR + mid * LANES, LANES)][0]
                lt = jnp.logical_and(active, v < x)
                ge = jnp.logical_and(active, jnp.logical_not(v < x))
                lo_b = jnp.where(lt, mid + 1, lo_b)
                hi_b = jnp.where(ge, mid, hi_b)
                return lo_b, hi_b

            m, _ = lax.fori_loop(0, WSTEPS, wbisect,
                                 (jnp.int32(0), jnp.int32(SSTR // LANES)))
            mo = jnp.maximum(m - 1, 0)
            v = win[pl.ds(s * SSTR + mo * LANES, LANES)]
            cnt = jnp.int32(0)
            for l in range(LANES):
                cnt = cnt + jnp.where(v[l] < x, 1, 0)
            bnd[s] = bnd[s] + mo * LANES + cnt
            return 0

        fire_body(0, 0)
        fire_body(SPW, 0)
        drain_body(0, 0)
        drain_body(SPW, 0)
        refine_body(0, 0)
        refine_body(SPW, 0)

        lo = bnd[0]
        hi = bnd[SPW]

        k0 = lo // C
        k1 = (hi + (C - 1)) // C

        def start(k, half):
            base = jnp.minimum(k * C, N - C)
            for h, b in zip((h0, h1, h2), bufs):
                pltpu.async_copy(h.at[pl.ds(base, C)], b.at[half], sems[half])

        def wait(half):
            for h, b in zip((h0, h1, h2), bufs):
                pltpu.make_async_copy(h.at[pl.ds(0, C)], b.at[half],
                                      sems[half]).wait()

        for slot in range(4):
            @pl.when(k0 + slot < k1)
            def _():
                start(k0 + slot, slot)

        lax.fori_loop(1, SPW, fire_body, 0)
        lax.fori_loop(1, SPW, drain_body, 0)
        lax.fori_loop(1, SPW, refine_body, 0)

        zero16 = jnp.zeros((LANES,), jnp.float32)

        def zero_body(i, _):
            for ch in range(NCH):
                acc[i, pl.ds(ch * LANES, LANES)] = zero16
            return 0

        lax.fori_loop(0, SPW, zero_body, 0)

        zeros48 = tuple(zero16 for _ in range(NCH))

        def first_above(off, limit):
            def bisect(_, carry):
                lo_b, hi_b = carry
                active = lo_b < hi_b
                mid = (lo_b + hi_b) // 2
                le = jnp.logical_and(active, bnd[mid + off] <= limit)
                gt = jnp.logical_and(active,
                                     jnp.logical_not(bnd[mid + off] <= limit))
                lo_b = jnp.where(le, mid + 1, lo_b)
                hi_b = jnp.where(gt, mid, hi_b)
                return lo_b, hi_b

            s_b, _ = lax.fori_loop(0, 6, bisect,
                                   (jnp.int32(0), jnp.int32(SPW)))
            return s_b

        def process(k, half):
            rbase = jnp.minimum(k * C, N - C)
            r0 = jnp.maximum(lo, k * C)
            r1 = jnp.minimum(hi, k * C + C)

            s_first = first_above(1, r0)
            s_end = first_above(0, jnp.maximum(r1 - 1, r0))

            def srun(s, _):
                ra = jnp.maximum(bnd[s], r0)
                rb = jnp.minimum(bnd[s + 1], r1)

                @plsc.parallel_loop(ra, rb, carry=zeros48)
                def run_sum(r, carry):
                    rr = r - rbase
                    vals = []
                    for j, b in enumerate(bufs):
                        for ch in range(DCH):
                            vals.append(b[half, rr, pl.ds(ch * LANES, LANES)])
                    return tuple(carry[i] + vals[i] for i in range(NCH))

                for i in range(NCH):
                    plsc.addupdate(acc.at[s, pl.ds(i * LANES, LANES)],
                                   run_sum[i])
                return 0

            lax.fori_loop(s_first, s_end, srun, 0)

        def ring_body(q, _):
            for slot in (0, 1, 2, 3):
                k = k0 + 4 * q + slot

                @pl.when(k < k1)
                def _():
                    wait(slot)
                    process(k, slot)

                    @pl.when(k + 4 < k1)
                    def _():
                        start(k + 4, slot)
            return 0

        lax.fori_loop(0, (k1 - k0 + 3) // 4, ring_body, 0)

        def fin_body(s, _):
            cf = (bnd[s + 1] - bnd[s]).astype(jnp.float32)
            cvec = lax.broadcast_in_dim(cf, (LANES,), ())
            inv = 1.0 / jnp.maximum(cvec, 1.0)
            for ch in range(NCH):
                acc[s, pl.ds(ch * LANES, LANES)] = (
                    acc[s, pl.ds(ch * LANES, LANES)] * inv)
            return 0

        lax.fori_loop(0, SPW, fin_body, 0)
        pltpu.sync_copy(acc, out.at[pl.ds(seg_lo, SPW)])

    mesh = plsc.VectorSubcoreMesh(
        core_axis_name="c", subcore_axis_name="s",
        num_cores=NC, num_subcores=NS)
    return pl.kernel(
        body,
        out_type=jax.ShapeDtypeStruct((NSEG, DF), jnp.float32),
        mesh=mesh,
        scratch_types=[
            pltpu.VMEM((NSAMP + 16, ), jnp.int32),
            pltpu.VMEM(((SPW + 2) * SSTR,), jnp.int32),
            pltpu.VMEM((4, C, D), jnp.float32),
            pltpu.VMEM((4, C, D), jnp.float32),
            pltpu.VMEM((4, C, D), jnp.float32),
            pltpu.VMEM((NSEG // NW, 3 * D), jnp.float32),
            pltpu.SMEM((48,), jnp.int32),
            pltpu.SemaphoreType.DMA,
            pltpu.SemaphoreType.DMA,
            pltpu.SemaphoreType.DMA,
            pltpu.SemaphoreType.DMA,
            pltpu.SemaphoreType.DMA,
        ],
        interpret=interpret,
    )


def kernel(h0, h1, h2, index):
    N, D = h0.shape
    idx = index.astype(jnp.int32)
    samp = idx[::64]
    fn = _make_sc_kernel(N, D, C=32)
    return fn(h0, h1, h2, idx, samp)


# speedup vs baseline: 1.0184x; 1.0184x over previous
import jax
import jax.numpy as jnp
from jax import lax
from jax.experimental import pallas as pl
from jax.experimental.pallas import tpu as pltpu
from jax.experimental.pallas import tpu_sc as plsc

NSEG = 1024
LANES = 16
NC = 2
NS = 16
NW = NC * NS


def _make_sc_kernel(N, D, C, interpret=False):
    SPW = NSEG // NW
    DF = 3 * D
    NCH = DF // LANES
    DCH = D // LANES
    SSTR = 256
    NSAMP = (N + SSTR - 1) // SSTR
    CSTEPS = NSAMP.bit_length()
    WSTEPS = (SSTR // LANES).bit_length()
    assert N % LANES == 0 and C % 8 == 0 and (N - C) % 8 == 0

    def body(h0, h1, h2, idxh, samph, out,
             samp, win, b0, b1, b2, acc, bnd, sem0, sem1, sem2, semw):
        sems = (sem0, sem1, sem2)
        bufs = (b0, b1, b2)
        cid = lax.axis_index("c")
        sid = lax.axis_index("s")
        w = sid * NC + cid
        seg_lo = w * SPW

        pltpu.sync_copy(samph, samp.at[pl.ds(0, NSAMP)])

        def coarse(x):
            def bisect(_, carry):
                lo_b, hi_b = carry
                active = lo_b < hi_b
                mid = (lo_b + hi_b) // 2
                v = samp[pl.ds(mid, LANES)][0]
                lt = jnp.logical_and(active, v < x)
                ge = jnp.logical_and(active, jnp.logical_not(v < x))
                lo_b = jnp.where(lt, mid + 1, lo_b)
                hi_b = jnp.where(ge, mid, hi_b)
                return lo_b, hi_b

            p, _ = lax.fori_loop(0, CSTEPS, bisect,
                                 (jnp.int32(0), jnp.int32(NSAMP)))
            return jnp.minimum(jnp.maximum(p - 1, 0) * SSTR, N - SSTR)

        def fire_body(s, _):
            wb = coarse(seg_lo + s)
            bnd[s] = wb
            pltpu.async_copy(idxh.at[pl.ds(wb, SSTR)],
                             win.at[pl.ds(s * SSTR, SSTR)], semw)
            return 0

        def drain_body(s, _):
            pltpu.make_async_copy(idxh.at[pl.ds(0, SSTR)],
                                  win.at[pl.ds(s * SSTR, SSTR)], semw).wait()
            return 0

        def refine_body(s, _):
            x = seg_lo + s

            def wbisect(_, carry):
                lo_b, hi_b = carry
                active = lo_b < hi_b
                mid = (lo_b + hi_b) // 2
                v = win[pl.ds(s * SSTR + mid * LANES, LANES)][0]
                lt = jnp.logical_and(active, v < x)
                ge = jnp.logical_and(active, jnp.logical_not(v < x))
                lo_b = jnp.where(lt, mid + 1, lo_b)
                hi_b = jnp.where(ge, mid, hi_b)
                return lo_b, hi_b

            m, _ = lax.fori_loop(0, WSTEPS, wbisect,
                                 (jnp.int32(0), jnp.int32(SSTR // LANES)))
            mo = jnp.maximum(m - 1, 0)
            v = win[pl.ds(s * SSTR + mo * LANES, LANES)]
            cnt = jnp.int32(0)
            for l in range(LANES):
                cnt = cnt + jnp.where(v[l] < x, 1, 0)
            bnd[s] = bnd[s] + mo * LANES + cnt
            return 0

        fire_body(0, 0)
        fire_body(SPW, 0)
        drain_body(0, 0)
        drain_body(SPW, 0)
        refine_body(0, 0)
        refine_body(SPW, 0)

        lo = bnd[0]
        hi = bnd[SPW]

        k0 = lo // C
        k1 = (hi + (C - 1)) // C

        def start(k, half):
            base = jnp.minimum(k * C, N - C)
            for h, b in zip((h0, h1, h2), bufs):
                pltpu.async_copy(h.at[pl.ds(base, C)], b.at[half], sems[half])

        def wait(half):
            for h, b in zip((h0, h1, h2), bufs):
                pltpu.make_async_copy(h.at[pl.ds(0, C)], b.at[half],
                                      sems[half]).wait()

        for slot in range(3):
            @pl.when(k0 + slot < k1)
            def _():
                start(k0 + slot, slot)

        lax.fori_loop(1, SPW, fire_body, 0)
        lax.fori_loop(1, SPW, drain_body, 0)
        lax.fori_loop(1, SPW, refine_body, 0)

        zero16 = jnp.zeros((LANES,), jnp.float32)

        def zero_body(i, _):
            for ch in range(NCH):
                acc[i, pl.ds(ch * LANES, LANES)] = zero16
            return 0

        lax.fori_loop(0, SPW, zero_body, 0)

        zeros48 = tuple(zero16 for _ in range(NCH))

        def first_above(off, limit):
            def bisect(_, carry):
                lo_b, hi_b = carry
                active = lo_b < hi_b
                mid = (lo_b + hi_b) // 2
                le = jnp.logical_and(active, bnd[mid + off] <= limit)
                gt = jnp.logical_and(active,
                                     jnp.logical_not(bnd[mid + off] <= limit))
                lo_b = jnp.where(le, mid + 1, lo_b)
                hi_b = jnp.where(gt, mid, hi_b)
                return lo_b, hi_b

            s_b, _ = lax.fori_loop(0, 6, bisect,
                                   (jnp.int32(0), jnp.int32(SPW)))
            return s_b

        def process(k, half):
            rbase = jnp.minimum(k * C, N - C)
            r0 = jnp.maximum(lo, k * C)
            r1 = jnp.minimum(hi, k * C + C)

            s_first = first_above(1, r0)
            s_end = first_above(0, jnp.maximum(r1 - 1, r0))

            def srun(s, _):
                ra = jnp.maximum(bnd[s], r0)
                rb = jnp.minimum(bnd[s + 1], r1)

                @plsc.parallel_loop(ra, rb, carry=zeros48)
                def run_sum(r, carry):
                    rr = r - rbase
                    vals = []
                    for j, b in enumerate(bufs):
                        for ch in range(DCH):
                            vals.append(b[half, rr, pl.ds(ch * LANES, LANES)])
                    return tuple(carry[i] + vals[i] for i in range(NCH))

                for i in range(NCH):
                    plsc.addupdate(acc.at[s, pl.ds(i * LANES, LANES)],
                                   run_sum[i])
                return 0

            lax.fori_loop(s_first, s_end, srun, 0)

        def ring_body(q, _):
            for slot in (0, 1, 2):
                k = k0 + 3 * q + slot

                @pl.when(k < k1)
                def _():
                    wait(slot)
                    process(k, slot)

                    @pl.when(k + 3 < k1)
                    def _():
                        start(k + 3, slot)
            return 0

        lax.fori_loop(0, (k1 - k0 + 2) // 3, ring_body, 0)

        def fin_body(s, _):
            cf = (bnd[s + 1] - bnd[s]).astype(jnp.float32)
            cvec = lax.broadcast_in_dim(cf, (LANES,), ())
            inv = 1.0 / jnp.maximum(cvec, 1.0)
            for ch in range(NCH):
                acc[s, pl.ds(ch * LANES, LANES)] = (
                    acc[s, pl.ds(ch * LANES, LANES)] * inv)
            return 0

        lax.fori_loop(0, SPW, fin_body, 0)
        pltpu.sync_copy(acc, out.at[pl.ds(seg_lo, SPW)])

    mesh = plsc.VectorSubcoreMesh(
        core_axis_name="c", subcore_axis_name="s",
        num_cores=NC, num_subcores=NS)
    return pl.kernel(
        body,
        out_type=jax.ShapeDtypeStruct((NSEG, DF), jnp.float32),
        mesh=mesh,
        scratch_types=[
            pltpu.VMEM((NSAMP + 16, ), jnp.int32),
            pltpu.VMEM(((SPW + 2) * SSTR,), jnp.int32),
            pltpu.VMEM((3, C, D), jnp.float32),
            pltpu.VMEM((3, C, D), jnp.float32),
            pltpu.VMEM((3, C, D), jnp.float32),
            pltpu.VMEM((NSEG // NW, 3 * D), jnp.float32),
            pltpu.SMEM((48,), jnp.int32),
            pltpu.SemaphoreType.DMA,
            pltpu.SemaphoreType.DMA,
            pltpu.SemaphoreType.DMA,
            pltpu.SemaphoreType.DMA,
        ],
        interpret=interpret,
    )


def kernel(h0, h1, h2, index):
    N, D = h0.shape
    idx = index.astype(jnp.int32)
    samp = idx[::256]
    fn = _make_sc_kernel(N, D, C=40)
    return fn(h0, h1, h2, idx, samp)
